# Initial kernel scaffold; baseline (speedup 1.0000x reference)
import jax
import jax.numpy as jnp
from jax.experimental import pallas as pl


def kernel(x, chem_polar, net_charge, hydropathy, mol_mass, ang, mask_aa, token_embed, atom_mask_embed, chem_polar_embed, net_charge_embed, hydropathy_W, mol_mass_W, angle_W):
    raise NotImplementedError("write your pallas kernel here")


# speedup vs baseline: 3.0581x; 3.0581x over previous
import functools

import jax
import jax.numpy as jnp
from jax import lax
from jax.experimental import pallas as pl
from jax.experimental.pallas import tpu as pltpu


_T = 2048


def _body(x_ref, cp_ref, nc_ref, hyd_ref, mol_ref, a0_ref, a1_ref, a2_ref,
          m_ref, w_ref, misc_ref, am_ref, out_ref):
    T = x_ref.shape[0]
    xi = x_ref[...]
    cpi = cp_ref[...]
    nci = nc_ref[...]
    iota = lax.broadcasted_iota(jnp.int32, (T, 48), 1)
    oh = ((iota == xi) | (iota == cpi + 32) | (iota == nci + 39))
    ohf = oh.astype(jnp.float32)
    h = jnp.dot(ohf, w_ref[...], preferred_element_type=jnp.float32)

    misc = misc_ref[...]
    h = h + hyd_ref[...] * misc[0:1, :]
    h = h + mol_ref[...] * misc[1:2, :]
    for j, a_ref in enumerate((a0_ref, a1_ref, a2_ref)):
        a = a_ref[...] / 180.0
        a = jnp.where(jnp.isinf(a), 0.0, a)
        h = h + a * misc[2 + j:3 + j, :]

    mrow = jnp.sum(am_ref[...], axis=0, keepdims=True)
    m = m_ref[...] != 0
    out_ref[...] = jnp.where(m, mrow, h)


@functools.partial(jax.jit, static_argnames=())
def kernel(x, chem_polar, net_charge, hydropathy, mol_mass, ang, mask_aa,
           token_embed, atom_mask_embed, chem_polar_embed, net_charge_embed,
           hydropathy_W, mol_mass_W, angle_W):
    B, L = x.shape
    H = token_embed.shape[1]
    N = B * L
    T = _T
    G = N // T

    col = lambda v: v.reshape(N, 1)
    xi = col(x.astype(jnp.int32))
    cpi = col(chem_polar.astype(jnp.int32))
    nci = col(net_charge.astype(jnp.int32))
    hyd = col(hydropathy)
    mol = col(mol_mass)
    a0 = col(ang[..., 0])
    a1 = col(ang[..., 1])
    a2 = col(ang[..., 2])
    m = col(mask_aa)

    w = jnp.zeros((48, H), jnp.float32)
    w = w.at[0:32].set(token_embed)
    w = w.at[32:39].set(chem_polar_embed)
    w = w.at[39:43].set(net_charge_embed)
    misc = jnp.concatenate(
        [hydropathy_W.T, mol_mass_W.T, angle_W.T,
         jnp.zeros((3, H), jnp.float32)], axis=0)

    tok_spec = pl.BlockSpec((T, 1), lambda i: (i, 0))
    full = lambda s: pl.BlockSpec(s, lambda i: (0, 0))

    out = pl.pallas_call(
        _body,
        grid=(G,),
        in_specs=[tok_spec] * 9 + [full((48, H)), full((8, H)),
                                   full((9, H))],
        out_specs=pl.BlockSpec((T, H), lambda i: (i, 0)),
        out_shape=jax.ShapeDtypeStruct((N, H), jnp.float32),
        compiler_params=pltpu.CompilerParams(
            dimension_semantics=("arbitrary",)),
    )(xi, cpi, nci, hyd, mol, a0, a1, a2, m, w, misc, atom_mask_embed)
    return out.reshape(B, L, H)


# speedup vs baseline: 36.6549x; 11.9861x over previous
import jax
import jax.numpy as jnp
from jax import lax
from jax.experimental import pallas as pl
from jax.experimental.pallas import tpu as pltpu


_R = 8


def _body(x_ref, cp_ref, nc_ref, hyd_ref, mol_ref, a0_ref, a1_ref, a2_ref,
          m_ref, w_ref, out_ref):
    R, L = x_ref.shape
    w = w_ref[...]
    iota = lax.broadcasted_iota(jnp.int32, (43, L), 0)
    for r in range(R):
        xi = x_ref[r:r + 1, :]
        cpi = cp_ref[r:r + 1, :]
        nci = nc_ref[r:r + 1, :]
        oh = ((iota == xi) | (iota == cpi + 32) | (iota == nci + 39))
        m = (m_ref[r:r + 1, :] != 0).astype(jnp.float32)
        notm = 1.0 - m
        ohf = oh.astype(jnp.float32) * notm

        def angrow(a_ref):
            a = a_ref[r:r + 1, :] / 180.0
            return jnp.where(jnp.isinf(a), 0.0, a) * notm

        a_mat = jnp.concatenate(
            [ohf,
             hyd_ref[r:r + 1, :] * notm,
             mol_ref[r:r + 1, :] * notm,
             angrow(a0_ref), angrow(a1_ref), angrow(a2_ref),
             jnp.broadcast_to(m, (9, L)),
             jnp.zeros((7, L), jnp.float32)], axis=0)

        h = lax.dot_general(a_mat, w, (((0,), (0,)), ((), ())),
                            preferred_element_type=jnp.float32)
        out_ref[pl.ds(r * L, L), :] = h


def kernel(x, chem_polar, net_charge, hydropathy, mol_mass, ang, mask_aa,
           token_embed, atom_mask_embed, chem_polar_embed, net_charge_embed,
           hydropathy_W, mol_mass_W, angle_W):
    B, L = x.shape
    H = token_embed.shape[1]
    N = B * L
    R = _R
    G = B // R

    w = jnp.concatenate(
        [token_embed,
         chem_polar_embed,
         net_charge_embed,
         hydropathy_W.T,
         mol_mass_W.T,
         angle_W.T,
         atom_mask_embed,
         jnp.zeros((7, H), jnp.float32)], axis=0)

    tok = pl.BlockSpec((R, L), lambda i: (i, 0))

    out = pl.pallas_call(
        _body,
        grid=(G,),
        in_specs=[tok] * 9 + [pl.BlockSpec((64, H), lambda i: (0, 0))],
        out_specs=pl.BlockSpec((R * L, H), lambda i: (i, 0)),
        out_shape=jax.ShapeDtypeStruct((N, H), jnp.float32),
        compiler_params=pltpu.CompilerParams(
            dimension_semantics=("arbitrary",)),
    )(x.astype(jnp.int32), chem_polar.astype(jnp.int32),
      net_charge.astype(jnp.int32),
      hydropathy[..., 0], mol_mass[..., 0],
      ang[..., 0], ang[..., 1], ang[..., 2],
      mask_aa[..., 0], w)
    return out.reshape(B, L, H)
